# baseline (speedup 1.0000x reference)
import functools

import jax
import jax.numpy as jnp
import numpy as np
from jax import lax
from jax.experimental import pallas as pl
from jax.experimental.pallas import tpu as pltpu
from jax.experimental.pallas import tpu_sc as plsc

B = 16384
D = 64
NC = 2
NS = 16
NW = NC * NS
BPW = B // NW
CHUNK = 128
NGRP = CHUNK // 16
NCHUNK = BPW // CHUNK
WAVE = 32
NWAVE = CHUNK // WAVE

_EPS = np.float32(1e-12)


def _inv_norm(s):
    sc = jnp.maximum(s, np.float32(1e-30))
    i = lax.bitcast_convert_type(sc, jnp.int32)
    i = np.int32(0x5F3759DF) - lax.shift_right_logical(i, 1)
    y = lax.bitcast_convert_type(i, jnp.float32)
    half = np.float32(0.5) * sc
    for _ in range(3):
        y = y * (np.float32(1.5) - half * y * y)
    norm = sc * y
    return np.float32(1.0) / jnp.maximum(norm, _EPS)


@functools.partial(
    pl.kernel,
    out_type=jax.ShapeDtypeStruct((B,), jnp.float32),
    mesh=plsc.VectorSubcoreMesh(core_axis_name="c", subcore_axis_name="s"),
    compiler_params=pltpu.CompilerParams(
        use_tc_tiling_on_sc=False, needs_layout_passes=False
    ),
    scratch_types=[
        pltpu.SMEM((BPW,), jnp.int32),
        pltpu.SMEM((BPW,), jnp.int32),
        pltpu.SMEM((BPW,), jnp.int32),
        pltpu.VMEM((BPW,), jnp.int32),
        pltpu.VMEM_SHARED((NS, BPW), jnp.int32),
        pltpu.VMEM_SHARED((NS, 4 * CHUNK, D), jnp.float32),
        pltpu.VMEM((CHUNK, D), jnp.float32),
        pltpu.VMEM((CHUNK, D), jnp.float32),
        pltpu.VMEM((CHUNK, D), jnp.float32),
        pltpu.VMEM((CHUNK, D), jnp.float32),
        pltpu.VMEM((16 * D,), jnp.float32),
        pltpu.VMEM((BPW,), jnp.float32),
        pltpu.SemaphoreType.DMA,
    ],
)
def _transh_sc(h_idx_hbm, r_idx_hbm, t_idx_hbm, ent_hbm, rel_hbm, nv_hbm,
               out_hbm, idx_h, idx_r, idx_t, idx_v, idx_sh, rows_sh, hb, tb,
               rb, nb, nscr, outb, sem):
    wid = lax.axis_index("s") * NC + lax.axis_index("c")
    sid = lax.axis_index("s")
    base = wid * BPW
    for src, dst in ((h_idx_hbm, idx_h), (r_idx_hbm, idx_r),
                     (t_idx_hbm, idx_t)):
        pltpu.sync_copy(src.at[pl.ds(base, BPW)], idx_v)
        pltpu.sync_copy(idx_v, idx_sh.at[sid])
        pltpu.sync_copy(idx_sh.at[sid], dst)

    zeros = jnp.zeros((16,), jnp.float32)
    lane = lax.iota(jnp.int32, 16)

    def group_body(g, cb):
        rows = g * 16 + lane

        s_n = zeros
        for d in range(D):
            col = jnp.full((16,), d, jnp.int32)
            v = plsc.load_gather(nb, [rows, col])
            s_n = s_n + v * v
        inv_n = _inv_norm(s_n)

        hn = zeros
        tn = zeros
        sh = zeros
        st = zeros
        sr = zeros
        for d in range(D):
            col = jnp.full((16,), d, jnp.int32)
            nd = plsc.load_gather(nb, [rows, col]) * inv_n
            nscr[pl.ds(d * 16, 16)] = nd
            hd = plsc.load_gather(hb, [rows, col])
            td = plsc.load_gather(tb, [rows, col])
            rd = plsc.load_gather(rb, [rows, col])
            hn = hn + hd * nd
            tn = tn + td * nd
            sh = sh + hd * hd
            st = st + td * td
            sr = sr + rd * rd
        shp = jnp.maximum(sh - hn * hn, np.float32(0.0))
        stp = jnp.maximum(st - tn * tn, np.float32(0.0))
        ih = _inv_norm(shp)
        it = _inv_norm(stp)
        ir = _inv_norm(sr)

        sc = zeros
        for d in range(D):
            col = jnp.full((16,), d, jnp.int32)
            nd = nscr[pl.ds(d * 16, 16)]
            hd = plsc.load_gather(hb, [rows, col])
            td = plsc.load_gather(tb, [rows, col])
            rd = plsc.load_gather(rb, [rows, col])
            hh = (hd - hn * nd) * ih
            tt = (td - tn * nd) * it
            rr = rd * ir
            sc = sc + jnp.abs(hh + rr - tt)
        outb[pl.ds(cb + g * 16, 16)] = sc
        return cb

    def issue_wave(w, cb):
        def body(r, _):
            rr = w * WAVE + r
            pltpu.async_copy(ent_hbm.at[idx_h[cb + rr]],
                             rows_sh.at[sid, rr], sem)
            pltpu.async_copy(ent_hbm.at[idx_t[cb + rr]],
                             rows_sh.at[sid, CHUNK + rr], sem)
            pltpu.async_copy(rel_hbm.at[idx_r[cb + rr]],
                             rows_sh.at[sid, 2 * CHUNK + rr], sem)
            pltpu.async_copy(nv_hbm.at[idx_r[cb + rr]],
                             rows_sh.at[sid, 3 * CHUNK + rr], sem)
            return 0

        lax.fori_loop(0, WAVE, body, 0)

    def drain_wave():
        def body(r, _):
            for _u in range(4):
                pltpu.make_async_copy(ent_hbm.at[0], rows_sh.at[0, 0],
                                      sem).wait()
            return 0

        lax.fori_loop(0, WAVE, body, 0)

    def chunk_body(c, _):
        cb = pl.multiple_of(c * CHUNK, CHUNK)
        issue_wave(0, cb)
        for w in range(1, NWAVE):
            issue_wave(w, cb)
            drain_wave()
        drain_wave()
        pltpu.sync_copy(rows_sh.at[sid, pl.ds(0, CHUNK)], hb)
        pltpu.sync_copy(rows_sh.at[sid, pl.ds(CHUNK, CHUNK)], tb)
        pltpu.sync_copy(rows_sh.at[sid, pl.ds(2 * CHUNK, CHUNK)], rb)
        pltpu.sync_copy(rows_sh.at[sid, pl.ds(3 * CHUNK, CHUNK)], nb)
        lax.fori_loop(0, NGRP, group_body, cb)
        return 0

    lax.fori_loop(0, NCHUNK, chunk_body, 0)
    pltpu.sync_copy(outb, out_hbm.at[pl.ds(base, BPW)])


def kernel(triplet_idx, entity_emb, relation_emb, norm_vec):
    h_idx = triplet_idx[:, 0]
    r_idx = triplet_idx[:, 1]
    t_idx = triplet_idx[:, 2]
    return _transh_sc(h_idx, r_idx, t_idx, entity_emb, relation_emb,
                      norm_vec)


# speedup vs baseline: 1.0635x; 1.0635x over previous
import functools

import jax
import jax.numpy as jnp
import numpy as np
from jax import lax
from jax.experimental import pallas as pl
from jax.experimental.pallas import tpu as pltpu
from jax.experimental.pallas import tpu_sc as plsc

B = 16384
D = 64
NC = 2
NS = 16
NW = NC * NS
BPW = B // NW
CHUNK = 128
NGRP = CHUNK // 16
NCHUNK = BPW // CHUNK

_EPS = np.float32(1e-12)


def _inv_norm(s):
    sc = jnp.maximum(s, np.float32(1e-30))
    i = lax.bitcast_convert_type(sc, jnp.int32)
    i = np.int32(0x5F3759DF) - lax.shift_right_logical(i, 1)
    y = lax.bitcast_convert_type(i, jnp.float32)
    half = np.float32(0.5) * sc
    for _ in range(3):
        y = y * (np.float32(1.5) - half * y * y)
    norm = sc * y
    return np.float32(1.0) / jnp.maximum(norm, _EPS)


@functools.partial(
    pl.kernel,
    out_type=jax.ShapeDtypeStruct((B,), jnp.float32),
    mesh=plsc.VectorSubcoreMesh(core_axis_name="c", subcore_axis_name="s"),
    compiler_params=pltpu.CompilerParams(
        use_tc_tiling_on_sc=False, needs_layout_passes=False
    ),
    scratch_types=[
        pltpu.VMEM((BPW,), jnp.int32),
        pltpu.VMEM((BPW,), jnp.int32),
        pltpu.VMEM((BPW,), jnp.int32),
        pltpu.VMEM((CHUNK, D), jnp.float32),
        pltpu.VMEM((CHUNK, D), jnp.float32),
        pltpu.VMEM((CHUNK, D), jnp.float32),
        pltpu.VMEM((CHUNK, D), jnp.float32),
        pltpu.VMEM((CHUNK, D), jnp.float32),
        pltpu.VMEM((CHUNK, D), jnp.float32),
        pltpu.VMEM((CHUNK, D), jnp.float32),
        pltpu.VMEM((CHUNK, D), jnp.float32),
        pltpu.VMEM((16 * D,), jnp.float32),
        pltpu.VMEM((BPW,), jnp.float32),
        pltpu.SemaphoreType.DMA,
        pltpu.SemaphoreType.DMA,
    ],
)
def _transh_sc(h_idx_hbm, r_idx_hbm, t_idx_hbm, ent_hbm, rel_hbm, nv_hbm,
               out_hbm, idx_h, idx_r, idx_t, hb0, tb0, rb0, nb0, hb1, tb1,
               rb1, nb1, nscr, outb, sem_a, sem_b):
    wid = lax.axis_index("s") * NC + lax.axis_index("c")
    base = wid * BPW
    pltpu.sync_copy(h_idx_hbm.at[pl.ds(base, BPW)], idx_h)
    pltpu.sync_copy(r_idx_hbm.at[pl.ds(base, BPW)], idx_r)
    pltpu.sync_copy(t_idx_hbm.at[pl.ds(base, BPW)], idx_t)

    zeros = jnp.zeros((16,), jnp.float32)
    lane = lax.iota(jnp.int32, 16)

    def issue(cb, bufs, sem):
        hb, tb, rb, nb = bufs
        for gi in range(NGRP):
            off = cb + gi * 16
            dst = pl.ds(gi * 16, 16)
            iv_h = idx_h[pl.ds(off, 16)]
            iv_t = idx_t[pl.ds(off, 16)]
            iv_r = idx_r[pl.ds(off, 16)]
            pltpu.async_copy(ent_hbm.at[iv_h], hb.at[dst], sem)
            pltpu.async_copy(ent_hbm.at[iv_t], tb.at[dst], sem)
            pltpu.async_copy(rel_hbm.at[iv_r], rb.at[dst], sem)
            pltpu.async_copy(nv_hbm.at[iv_r], nb.at[dst], sem)

    def drain(bufs, sem):
        hb, tb, rb, nb = bufs
        src = ent_hbm.at[pl.ds(0, 16)]
        for gi in range(NGRP):
            dst = pl.ds(gi * 16, 16)
            pltpu.make_async_copy(src, hb.at[dst], sem).wait()
            pltpu.make_async_copy(src, tb.at[dst], sem).wait()
            pltpu.make_async_copy(src, rb.at[dst], sem).wait()
            pltpu.make_async_copy(src, nb.at[dst], sem).wait()

    def make_compute(bufs):
        hb, tb, rb, nb = bufs

        def group_body(g, cb):
            rows = g * 16 + lane

            s_n = zeros
            for d in range(D):
                col = jnp.full((16,), d, jnp.int32)
                v = plsc.load_gather(nb, [rows, col])
                s_n = s_n + v * v
            inv_n = _inv_norm(s_n)

            hn = zeros
            tn = zeros
            sh = zeros
            st = zeros
            sr = zeros
            for d in range(D):
                col = jnp.full((16,), d, jnp.int32)
                nd = plsc.load_gather(nb, [rows, col]) * inv_n
                nscr[pl.ds(d * 16, 16)] = nd
                hd = plsc.load_gather(hb, [rows, col])
                td = plsc.load_gather(tb, [rows, col])
                rd = plsc.load_gather(rb, [rows, col])
                hn = hn + hd * nd
                tn = tn + td * nd
                sh = sh + hd * hd
                st = st + td * td
                sr = sr + rd * rd
            shp = jnp.maximum(sh - hn * hn, np.float32(0.0))
            stp = jnp.maximum(st - tn * tn, np.float32(0.0))
            ih = _inv_norm(shp)
            it = _inv_norm(stp)
            ir = _inv_norm(sr)

            sc = zeros
            for d in range(D):
                col = jnp.full((16,), d, jnp.int32)
                nd = nscr[pl.ds(d * 16, 16)]
                hd = plsc.load_gather(hb, [rows, col])
                td = plsc.load_gather(tb, [rows, col])
                rd = plsc.load_gather(rb, [rows, col])
                hh = (hd - hn * nd) * ih
                tt = (td - tn * nd) * it
                rr = rd * ir
                sc = sc + jnp.abs(hh + rr - tt)
            outb[pl.ds(cb + g * 16, 16)] = sc
            return cb

        def compute(cb):
            lax.fori_loop(0, NGRP, group_body, cb)

        return compute

    bufs0 = (hb0, tb0, rb0, nb0)
    bufs1 = (hb1, tb1, rb1, nb1)
    compute0 = make_compute(bufs0)
    compute1 = make_compute(bufs1)

    issue(0, bufs0, sem_a)

    def pair_body(i, _):
        cb0 = pl.multiple_of(2 * i * CHUNK, CHUNK)
        cb1 = pl.multiple_of((2 * i + 1) * CHUNK, CHUNK)
        issue(cb1, bufs1, sem_b)
        drain(bufs0, sem_a)
        compute0(cb0)

        @pl.when(i < NCHUNK // 2 - 1)
        def _():
            cb2 = pl.multiple_of((2 * i + 2) * CHUNK, CHUNK)
            issue(cb2, bufs0, sem_a)

        drain(bufs1, sem_b)
        compute1(cb1)
        return 0

    lax.fori_loop(0, NCHUNK // 2, pair_body, 0)
    pltpu.sync_copy(outb, out_hbm.at[pl.ds(base, BPW)])


def kernel(triplet_idx, entity_emb, relation_emb, norm_vec):
    h_idx = triplet_idx[:, 0]
    r_idx = triplet_idx[:, 1]
    t_idx = triplet_idx[:, 2]
    return _transh_sc(h_idx, r_idx, t_idx, entity_emb, relation_emb,
                      norm_vec)
